# baseline (speedup 1.0000x reference)
import jax
import jax.numpy as jnp
from jax import lax
from jax.experimental import pallas as pl
from jax.experimental.pallas import tpu as pltpu
from jax.experimental.pallas import tpu_sc as plsc

N = 10000
NP = 10240
D = 128
E = 320000
NW = 32
CH = 128
NCH = 80
EP = NW * NCH * CH
RPT = NP // 16
HNCH = NCH // 2
SEG = 40
NCH_C0 = 160
NCH_C1 = 0
C1_BASE = 16 * NCH_C0

_mesh = plsc.VectorSubcoreMesh(core_axis_name="core", subcore_axis_name="subcore")


def _deg_body(dst_hbm, ones_hbm, z1_hbm, deg_hbm, acc, dst_v, ones_v, zed_v):
    c = lax.axis_index("core")
    s = lax.axis_index("subcore")
    wid = s * 2 + c
    pltpu.sync_copy(ones_hbm, ones_v)
    pltpu.sync_copy(z1_hbm, zed_v)
    pltpu.sync_copy(dst_hbm.at[pl.ds(wid * NCH, NCH)], dst_v)
    pltpu.sync_copy(zed_v, acc.at[pl.ds(s * RPT, RPT)])
    plsc.subcore_barrier()

    @pl.loop(0, NCH)
    def _(j):
        pltpu.sync_copy(ones_v, acc.at[dst_v.at[j]], add=True)

    plsc.subcore_barrier()
    pltpu.sync_copy(acc.at[pl.ds(s * RPT, RPT)], deg_hbm.at[pl.ds(c * NP + s * RPT, RPT)])


def _deg_call(dst_p, ones_h, z1):
    return pl.kernel(
        _deg_body,
        out_type=jax.ShapeDtypeStruct((2 * NP,), jnp.float32),
        mesh=_mesh,
        scratch_types=[
            pltpu.VMEM_SHARED((NP,), jnp.float32),
            pltpu.VMEM((NCH, CH), jnp.int32),
            pltpu.VMEM((CH,), jnp.float32),
            pltpu.VMEM((RPT,), jnp.float32),
        ],
    )(dst_p, ones_h, z1)


def _agg_body(xs_hbm, src_hbm, dst_hbm, z2_hbm, out_hbm, acc,
              src_v, dst_v, rows0, rows1, gs0, gs1, ss0, ss1):
    c = lax.axis_index("core")
    s = lax.axis_index("subcore")
    wid = s * 2 + c
    pltpu.sync_copy(z2_hbm, rows0)
    for k in range(RPT // CH):
        pltpu.sync_copy(rows0, acc.at[pl.ds(s * RPT + k * CH, CH)])
    plsc.subcore_barrier()

    def run_segment(base):
        pltpu.sync_copy(src_hbm.at[pl.ds(base, SEG)], src_v)
        pltpu.sync_copy(dst_hbm.at[pl.ds(base, SEG)], dst_v)
        pltpu.async_copy(xs_hbm.at[src_v.at[0]], rows0, gs0)
        pltpu.async_copy(xs_hbm.at[src_v.at[1]], rows1, gs1)

        @pl.loop(0, SEG, step=2)
        def _(j):
            pltpu.make_async_copy(xs_hbm.at[src_v.at[j]], rows0, gs0).wait()
            sc0 = pltpu.async_copy(rows0, acc.at[dst_v.at[j]], ss0, add=True)
            pltpu.make_async_copy(xs_hbm.at[src_v.at[j + 1]], rows1, gs1).wait()
            sc1 = pltpu.async_copy(rows1, acc.at[dst_v.at[j + 1]], ss1, add=True)

            @pl.when(j + 2 < SEG)
            def _():
                sc0.wait()
                pltpu.async_copy(xs_hbm.at[src_v.at[j + 2]], rows0, gs0)
                sc1.wait()
                pltpu.async_copy(xs_hbm.at[src_v.at[j + 3]], rows1, gs1)

            @pl.when(j + 2 >= SEG)
            def _():
                sc0.wait()
                sc1.wait()

    @pl.when(c == 0)
    def _():
        for seg in range(NCH_C0 // SEG):
            run_segment(s * NCH_C0 + seg * SEG)

    @pl.when(c == 1)
    def _():
        for seg in range(NCH_C1 // SEG):
            run_segment(C1_BASE + s * NCH_C1 + seg * SEG)

    plsc.subcore_barrier()
    pltpu.sync_copy(acc.at[pl.ds(s * RPT, RPT)], out_hbm.at[c, pl.ds(s * RPT, RPT)])


def _agg_call(xs, src_p, dst_p, z2):
    return pl.kernel(
        _agg_body,
        out_type=jax.ShapeDtypeStruct((2, NP, D), jnp.float32),
        mesh=_mesh,
        scratch_types=[
            pltpu.VMEM_SHARED((NP, D), jnp.float32),
            pltpu.VMEM((SEG, CH), jnp.int32),
            pltpu.VMEM((SEG, CH), jnp.int32),
            pltpu.VMEM((CH, D), jnp.float32),
            pltpu.VMEM((CH, D), jnp.float32),
            pltpu.SemaphoreType.DMA,
            pltpu.SemaphoreType.DMA,
            pltpu.SemaphoreType.DMA,
            pltpu.SemaphoreType.DMA,
        ],
    )(xs, src_p, dst_p, z2)


def _tc_a_body(x_ref, w_ref, degp_ref, xs_ref, dinv_ref):
    deg = degp_ref[0] + degp_ref[1] + 1.0
    dinv = lax.rsqrt(deg)
    dinv_ref[...] = dinv
    xw = jnp.dot(x_ref[...], w_ref[...], preferred_element_type=jnp.float32)
    xs_ref[...] = xw * dinv


def _tc_b1_body(s_ref, xs_ref, dinv_ref, b_ref, w_ref, xs2_ref):
    agg = s_ref[0] + s_ref[1] + xs_ref[...]
    h = jnp.maximum(agg * dinv_ref[...] + b_ref[...], 0.0)
    xw = jnp.dot(h, w_ref[...], preferred_element_type=jnp.float32)
    xs2_ref[...] = xw * dinv_ref[...]


def _tc_b2_body(s_ref, xs_ref, dinv_ref, b_ref, h_ref, xs3_ref):
    agg = s_ref[0] + s_ref[1] + xs_ref[...]
    h = jnp.maximum(agg * dinv_ref[...] + b_ref[...], 0.0)
    h_ref[...] = h
    xs3_ref[...] = h * dinv_ref[...]


def _tc_c_body(s_ref, xs_ref, dinv_ref, w_ref, b_ref, out_ref):
    agg = (s_ref[0] + s_ref[1] + xs_ref[...]) * dinv_ref[...]
    out_ref[...] = jnp.dot(agg, w_ref[...], preferred_element_type=jnp.float32) + b_ref[...]


def kernel(x, edge_index, W1, b1, W2, b2, W3, b3):
    src = edge_index[0].astype(jnp.int32)
    dst = edge_index[1].astype(jnp.int32)
    pad = EP - E
    src_p = jnp.concatenate([src, jnp.zeros((pad,), jnp.int32)]).reshape(NW * NCH, CH)
    dst_p = jnp.concatenate([dst, jnp.full((pad,), N, jnp.int32)]).reshape(NW * NCH, CH)
    x_p = jnp.pad(x, ((0, NP - N), (0, 0)))
    ones_h = jnp.ones((CH,), jnp.float32)
    z1 = jnp.zeros((RPT,), jnp.float32)
    z2 = jnp.zeros((CH, D), jnp.float32)

    deg_parts = _deg_call(dst_p, ones_h, z1)
    degp = deg_parts.reshape(2, NP, 1)

    xs1, dinv = pl.pallas_call(
        _tc_a_body,
        out_shape=[
            jax.ShapeDtypeStruct((NP, D), jnp.float32),
            jax.ShapeDtypeStruct((NP, 1), jnp.float32),
        ],
    )(x_p, W1, degp)

    s1 = _agg_call(xs1, src_p, dst_p, z2)

    xs2 = pl.pallas_call(
        _tc_b1_body,
        out_shape=jax.ShapeDtypeStruct((NP, D), jnp.float32),
    )(s1, xs1, dinv, b1.reshape(1, D), W2)

    s2 = _agg_call(xs2, src_p, dst_p, z2)

    h2, xs3 = pl.pallas_call(
        _tc_b2_body,
        out_shape=[
            jax.ShapeDtypeStruct((NP, D), jnp.float32),
            jax.ShapeDtypeStruct((NP, D), jnp.float32),
        ],
    )(s2, xs2, dinv, b2.reshape(1, D))

    s3 = _agg_call(xs3, src_p, dst_p, z2)

    out = pl.pallas_call(
        _tc_c_body,
        out_shape=jax.ShapeDtypeStruct((NP, W3.shape[1]), jnp.float32),
    )(s3, xs3, dinv, W3, b3.reshape(1, W3.shape[1]))

    return (out[:N], h2[:N])


# speedup vs baseline: 3.1899x; 3.1899x over previous
import jax
import jax.numpy as jnp
from jax import lax
from jax.experimental import pallas as pl
from jax.experimental.pallas import tpu as pltpu
from jax.experimental.pallas import tpu_sc as plsc

N = 10000
NP = 10240
D = 128
E = 320000
NW = 32
CH = 128
NCH = 80
EP = NW * NCH * CH
RPT = NP // 16
HNCH = NCH // 2
SEG = 40

_mesh = plsc.VectorSubcoreMesh(core_axis_name="core", subcore_axis_name="subcore")


def _deg_body(dst_hbm, ones_hbm, z1_hbm, deg_hbm, acc, dst_v, ones_v, zed_v):
    c = lax.axis_index("core")
    s = lax.axis_index("subcore")
    wid = s * 2 + c
    pltpu.sync_copy(ones_hbm, ones_v)
    pltpu.sync_copy(z1_hbm, zed_v)
    pltpu.sync_copy(dst_hbm.at[pl.ds(wid * NCH, NCH)], dst_v)
    pltpu.sync_copy(zed_v, acc.at[pl.ds(s * RPT, RPT)])
    plsc.subcore_barrier()

    @pl.loop(0, NCH)
    def _(j):
        pltpu.sync_copy(ones_v, acc.at[dst_v.at[j]], add=True)

    plsc.subcore_barrier()
    pltpu.sync_copy(acc.at[pl.ds(s * RPT, RPT)], deg_hbm.at[pl.ds(c * NP + s * RPT, RPT)])


def _deg_call(dst_p, ones_h, z1):
    return pl.kernel(
        _deg_body,
        out_type=jax.ShapeDtypeStruct((2 * NP,), jnp.float32),
        mesh=_mesh,
        scratch_types=[
            pltpu.VMEM_SHARED((NP,), jnp.float32),
            pltpu.VMEM((NCH, CH), jnp.int32),
            pltpu.VMEM((CH,), jnp.float32),
            pltpu.VMEM((RPT,), jnp.float32),
        ],
    )(dst_p, ones_h, z1)


def _agg_body(xs_hbm, src_hbm, dst_hbm, z2_hbm, out_hbm, acc,
              src_v, dst_v, rows0, rows1, gs0, gs1, ss0, ss1):
    c = lax.axis_index("core")
    s = lax.axis_index("subcore")
    wid = s * 2 + c
    pltpu.sync_copy(z2_hbm, rows0)
    for k in range(RPT // CH):
        pltpu.sync_copy(rows0, acc.at[pl.ds(s * RPT + k * CH, CH)])
    plsc.subcore_barrier()

    def run_segment(base):
        pltpu.sync_copy(src_hbm.at[pl.ds(base, SEG)], src_v)
        pltpu.sync_copy(dst_hbm.at[pl.ds(base, SEG)], dst_v)
        pltpu.async_copy(xs_hbm.at[src_v.at[0]], rows0, gs0)
        pltpu.async_copy(xs_hbm.at[src_v.at[1]], rows1, gs1)

        @pl.loop(0, SEG, step=2)
        def _(j):
            pltpu.make_async_copy(xs_hbm.at[src_v.at[j]], rows0, gs0).wait()
            sc0 = pltpu.async_copy(rows0, acc.at[dst_v.at[j]], ss0, add=True)
            pltpu.make_async_copy(xs_hbm.at[src_v.at[j + 1]], rows1, gs1).wait()
            sc1 = pltpu.async_copy(rows1, acc.at[dst_v.at[j + 1]], ss1, add=True)

            @pl.when(j + 2 < SEG)
            def _():
                sc0.wait()
                pltpu.async_copy(xs_hbm.at[src_v.at[j + 2]], rows0, gs0)
                sc1.wait()
                pltpu.async_copy(xs_hbm.at[src_v.at[j + 3]], rows1, gs1)

            @pl.when(j + 2 >= SEG)
            def _():
                sc0.wait()
                sc1.wait()

    for seg in range(NCH // SEG):
        run_segment(wid * NCH + seg * SEG)

    plsc.subcore_barrier()
    pltpu.sync_copy(acc.at[pl.ds(s * RPT, RPT)], out_hbm.at[c, pl.ds(s * RPT, RPT)])


def _agg_call(xs, src_p, dst_p, z2):
    return pl.kernel(
        _agg_body,
        out_type=jax.ShapeDtypeStruct((2, NP, D), jnp.float32),
        mesh=_mesh,
        scratch_types=[
            pltpu.VMEM_SHARED((NP, D), jnp.float32),
            pltpu.VMEM((SEG, CH), jnp.int32),
            pltpu.VMEM((SEG, CH), jnp.int32),
            pltpu.VMEM((CH, D), jnp.float32),
            pltpu.VMEM((CH, D), jnp.float32),
            pltpu.SemaphoreType.DMA,
            pltpu.SemaphoreType.DMA,
            pltpu.SemaphoreType.DMA,
            pltpu.SemaphoreType.DMA,
        ],
    )(xs, src_p, dst_p, z2)


def _tc_a_body(x_ref, w_ref, degp_ref, xs_ref, dinv_ref):
    deg = degp_ref[0] + degp_ref[1] + 1.0
    dinv = lax.rsqrt(deg)
    dinv_ref[...] = dinv
    xw = jnp.dot(x_ref[...], w_ref[...], preferred_element_type=jnp.float32)
    xs_ref[...] = xw * dinv


def _tc_b1_body(s_ref, xs_ref, dinv_ref, b_ref, w_ref, xs2_ref):
    agg = s_ref[0] + s_ref[1] + xs_ref[...]
    h = jnp.maximum(agg * dinv_ref[...] + b_ref[...], 0.0)
    xw = jnp.dot(h, w_ref[...], preferred_element_type=jnp.float32)
    xs2_ref[...] = xw * dinv_ref[...]


def _tc_b2_body(s_ref, xs_ref, dinv_ref, b_ref, h_ref, xs3_ref):
    agg = s_ref[0] + s_ref[1] + xs_ref[...]
    h = jnp.maximum(agg * dinv_ref[...] + b_ref[...], 0.0)
    h_ref[...] = h
    xs3_ref[...] = h * dinv_ref[...]


def _tc_c_body(s_ref, xs_ref, dinv_ref, w_ref, b_ref, out_ref):
    agg = (s_ref[0] + s_ref[1] + xs_ref[...]) * dinv_ref[...]
    out_ref[...] = jnp.dot(agg, w_ref[...], preferred_element_type=jnp.float32) + b_ref[...]


def kernel(x, edge_index, W1, b1, W2, b2, W3, b3):
    src = edge_index[0].astype(jnp.int32)
    dst = edge_index[1].astype(jnp.int32)
    pad = EP - E
    pad_idx = N + (jnp.arange(pad, dtype=jnp.int32) % (NP - N))
    src_p = jnp.concatenate([src, pad_idx]).reshape(NW * NCH, CH)
    dst_p = jnp.concatenate([dst, pad_idx]).reshape(NW * NCH, CH)
    x_p = jnp.pad(x, ((0, NP - N), (0, 0)))
    ones_h = jnp.ones((CH,), jnp.float32)
    z1 = jnp.zeros((RPT,), jnp.float32)
    z2 = jnp.zeros((CH, D), jnp.float32)

    deg_parts = _deg_call(dst_p, ones_h, z1)
    degp = deg_parts.reshape(2, NP, 1)

    xs1, dinv = pl.pallas_call(
        _tc_a_body,
        out_shape=[
            jax.ShapeDtypeStruct((NP, D), jnp.float32),
            jax.ShapeDtypeStruct((NP, 1), jnp.float32),
        ],
    )(x_p, W1, degp)

    s1 = _agg_call(xs1, src_p, dst_p, z2)

    xs2 = pl.pallas_call(
        _tc_b1_body,
        out_shape=jax.ShapeDtypeStruct((NP, D), jnp.float32),
    )(s1, xs1, dinv, b1.reshape(1, D), W2)

    s2 = _agg_call(xs2, src_p, dst_p, z2)

    h2, xs3 = pl.pallas_call(
        _tc_b2_body,
        out_shape=[
            jax.ShapeDtypeStruct((NP, D), jnp.float32),
            jax.ShapeDtypeStruct((NP, D), jnp.float32),
        ],
    )(s2, xs2, dinv, b2.reshape(1, D))

    s3 = _agg_call(xs3, src_p, dst_p, z2)

    out = pl.pallas_call(
        _tc_c_body,
        out_shape=jax.ShapeDtypeStruct((NP, W3.shape[1]), jnp.float32),
    )(s3, xs3, dinv, W3, b3.reshape(1, W3.shape[1]))

    return (out[:N], h2[:N])
